# baseline (speedup 1.0000x reference)
import functools
import math

import jax
import jax.numpy as jnp
import numpy as np
from jax import lax
from jax.experimental import pallas as pl
from jax.experimental.pallas import tpu as pltpu
from jax.experimental.pallas import tpu_sc as plsc

VOCAB = 100000
D_MODEL = 768
MAX_LEN = 4096
BATCH = 4
SCALE = math.sqrt(D_MODEL)

N_ROWS = MAX_LEN * BATCH
NW = 32
ROWS_PER_W = N_ROWS // NW
CHUNK_ROWS = 32
N_CHUNKS = ROWS_PER_W // CHUNK_ROWS
L_PER_CHUNK = CHUNK_ROWS // BATCH
LANES = 16
C_PER_ROW = D_MODEL // LANES


def _pe_table():
    pe = np.zeros((MAX_LEN, D_MODEL), dtype=np.float32)
    position = np.arange(0, MAX_LEN, dtype=np.float32)[:, None]
    div_term = np.exp(
        np.arange(0, D_MODEL, 2, dtype=np.float32) * (-math.log(10000.0) / D_MODEL)
    )
    pe[:, 0::2] = np.sin(position * div_term)
    pe[:, 1::2] = np.cos(position * div_term)
    u = pe.view(np.uint32)
    bf = ((u + 0x7FFF + ((u >> 16) & 1)) >> 16).astype(np.uint32)
    bb = bf.reshape(MAX_LEN, D_MODEL // 32, 2, LANES)
    packed = (bb[:, :, 1, :] << 16) | bb[:, :, 0, :]
    return jnp.asarray(packed.reshape(-1).view(np.int32))


_MESH = plsc.VectorSubcoreMesh(core_axis_name="c", subcore_axis_name="s")


@functools.partial(
    pl.kernel,
    mesh=_MESH,
    out_type=jax.ShapeDtypeStruct((MAX_LEN, BATCH, D_MODEL), jnp.float32),
    scratch_types=[
        pltpu.VMEM((ROWS_PER_W,), jnp.int32),
        pltpu.VMEM((2, CHUNK_ROWS, D_MODEL), jnp.float32),
        pltpu.VMEM((2, L_PER_CHUNK, BATCH, D_MODEL), jnp.float32),
        pltpu.VMEM((L_PER_CHUNK * D_MODEL // 2,), jnp.int32),
        pltpu.VMEM((L_PER_CHUNK * D_MODEL // 2,), jnp.int32),
        pltpu.SemaphoreType.DMA,
        pltpu.SemaphoreType.DMA,
        pltpu.SemaphoreType.DMA,
        pltpu.SemaphoreType.DMA,
        pltpu.SemaphoreType.DMA,
        pltpu.SemaphoreType.DMA,
    ],
)
def _embed_sc(table_hbm, idx_hbm, pe_hbm, out_hbm,
              idx_v, in_v, out_v, pe_v0, pe_v1,
              gsem0, gsem1, psem0, psem1, osem0, osem1):
    pe_vs = (pe_v0, pe_v1)
    wid = lax.axis_index("s") * 2 + lax.axis_index("c")
    base_row = wid * ROWS_PER_W
    base_l = wid * (ROWS_PER_W // BATCH)
    gsems = (gsem0, gsem1)
    psems = (psem0, psem1)
    osems = (osem0, osem1)

    pltpu.sync_copy(idx_hbm.at[pl.ds(wid * ROWS_PER_W, ROWS_PER_W)], idx_v)

    def start_in(g, s):
        l0 = base_l + g * L_PER_CHUNK
        pltpu.async_copy(
            pe_hbm.at[pl.ds(l0 * (D_MODEL // 2), L_PER_CHUNK * D_MODEL // 2)],
            pe_vs[s], psems[s])
        pltpu.async_copy(
            table_hbm.at[idx_v.at[pl.ds(g * CHUNK_ROWS, CHUNK_ROWS)]],
            in_v.at[s], gsems[s])

    def wait_in(s):
        pltpu.make_async_copy(
            pe_hbm.at[pl.ds(0, L_PER_CHUNK * D_MODEL // 2)],
            pe_vs[s], psems[s]).wait()
        pltpu.make_async_copy(
            table_hbm.at[pl.ds(0, CHUNK_ROWS)], in_v.at[s], gsems[s]).wait()


    def wait_out(s):
        pltpu.make_async_copy(
            out_hbm.at[pl.ds(0, L_PER_CHUNK)], out_v.at[s], osems[s]).wait()

    def do_chunk(g, s):
        @pl.when(g + 1 < N_CHUNKS)
        def _():
            start_in(g + 1, 1 - s)

        wait_in(s)

        @pl.when(g >= 2)
        def _():
            wait_out(s)

        @plsc.parallel_loop(0, L_PER_CHUNK)
        def _(li):
            base_r = li * BATCH
            n_pairs = C_PER_ROW // 2

            def load_pair(p):
                pe_i = pe_vs[s][pl.ds(li * (D_MODEL // 2) + p * LANES, LANES)]
                pe_a = lax.bitcast_convert_type(pe_i << 16, jnp.float32)
                pe_b = lax.bitcast_convert_type(
                    pe_i & jnp.int32(-65536), jnp.float32)
                sla = pl.ds(2 * p * LANES, LANES)
                slb = pl.ds((2 * p + 1) * LANES, LANES)
                return (pe_a, pe_b,
                        [in_v[s, base_r + b, sla] for b in range(BATCH)],
                        [in_v[s, base_r + b, slb] for b in range(BATCH)])

            grp = {p: load_pair(p) for p in range(2)}
            for p in range(n_pairs):
                pe_a, pe_b, ins_a, ins_b = grp.pop(p)
                if p + 2 < n_pairs:
                    grp[p + 2] = load_pair(p + 2)
                sla = pl.ds(2 * p * LANES, LANES)
                slb = pl.ds((2 * p + 1) * LANES, LANES)
                for b in range(BATCH):
                    out_v[s, li, b, sla] = ins_a[b] * SCALE + pe_a
                    out_v[s, li, b, slb] = ins_b[b] * SCALE + pe_b
        l0 = base_l + g * L_PER_CHUNK
        pltpu.async_copy(out_v.at[s], out_hbm.at[pl.ds(l0, L_PER_CHUNK)], osems[s])

    start_in(0, 0)

    def pair_body(i, carry):
        do_chunk(2 * i, 0)
        do_chunk(2 * i + 1, 1)
        return carry

    lax.fori_loop(0, N_CHUNKS // 2, pair_body, 0)
    wait_out(0)
    wait_out(1)


def kernel(encoded_words, embed_weight):
    idx = encoded_words.reshape(N_ROWS)
    pe = _pe_table()
    return _embed_sc(embed_weight, idx, pe)


# speedup vs baseline: 1.0717x; 1.0717x over previous
import functools
import math

import jax
import jax.numpy as jnp
import numpy as np
from jax import lax
from jax.experimental import pallas as pl
from jax.experimental.pallas import tpu as pltpu
from jax.experimental.pallas import tpu_sc as plsc

VOCAB = 100000
D_MODEL = 768
MAX_LEN = 4096
BATCH = 4
SCALE = math.sqrt(D_MODEL)

N_ROWS = MAX_LEN * BATCH
NW = 32
ROWS_PER_W = N_ROWS // NW
CHUNK_ROWS = 32
N_CHUNKS = ROWS_PER_W // CHUNK_ROWS
L_PER_CHUNK = CHUNK_ROWS // BATCH
LANES = 16
C_PER_ROW = D_MODEL // LANES


N_COARSE = MAX_LEN // L_PER_CHUNK


def _pack_bf16_pairs(arr):
    u = np.ascontiguousarray(arr, dtype=np.float32).view(np.uint32)
    bf = ((u + 0x7FFF + ((u >> 16) & 1)) >> 16).astype(np.uint32)
    bb = bf.reshape(arr.shape[0], D_MODEL // 32, 2, LANES)
    return ((bb[:, :, 1, :] << 16) | bb[:, :, 0, :]).reshape(
        arr.shape[0], D_MODEL // 2).view(np.int32)


def _pe_factors():
    w = np.exp(
        np.arange(0, D_MODEL, 2, dtype=np.float32) * (-math.log(10000.0) / D_MODEL)
    )
    a = np.arange(N_COARSE, dtype=np.float32)[:, None]
    d = np.arange(L_PER_CHUNK, dtype=np.float32)[:, None]
    X = np.zeros((N_COARSE, D_MODEL), np.float32)
    Y = np.zeros((N_COARSE, D_MODEL), np.float32)
    X[:, 0::2] = np.sin(L_PER_CHUNK * a * w)
    X[:, 1::2] = np.cos(L_PER_CHUNK * a * w)
    Y[:, 0::2] = np.cos(L_PER_CHUNK * a * w)
    Y[:, 1::2] = -np.sin(L_PER_CHUNK * a * w)
    U = np.zeros((L_PER_CHUNK, D_MODEL), np.float32)
    V = np.zeros((L_PER_CHUNK, D_MODEL), np.float32)
    U[:, 0::2] = np.cos(d * w)
    U[:, 1::2] = np.cos(d * w)
    V[:, 0::2] = np.sin(d * w)
    V[:, 1::2] = np.sin(d * w)
    xy = np.stack([_pack_bf16_pairs(X), _pack_bf16_pairs(Y)], axis=1)
    uv = np.concatenate(
        [_pack_bf16_pairs(U).reshape(-1), _pack_bf16_pairs(V).reshape(-1)])
    return jnp.asarray(xy.reshape(-1)), jnp.asarray(uv)


_MESH = plsc.VectorSubcoreMesh(core_axis_name="c", subcore_axis_name="s")


@functools.partial(
    pl.kernel,
    mesh=_MESH,
    out_type=jax.ShapeDtypeStruct((MAX_LEN, BATCH, D_MODEL), jnp.float32),
    scratch_types=[
        pltpu.VMEM((N_CHUNKS, CHUNK_ROWS), jnp.int32),
        pltpu.VMEM((2, CHUNK_ROWS, D_MODEL), jnp.float32),
        pltpu.VMEM((2, L_PER_CHUNK, BATCH, D_MODEL), jnp.float32),
        pltpu.VMEM((D_MODEL,), jnp.int32),
        pltpu.VMEM((D_MODEL,), jnp.int32),
        pltpu.VMEM((L_PER_CHUNK * D_MODEL,), jnp.int32),
        pltpu.SemaphoreType.DMA,
        pltpu.SemaphoreType.DMA,
        pltpu.SemaphoreType.DMA,
        pltpu.SemaphoreType.DMA,
        pltpu.SemaphoreType.DMA,
        pltpu.SemaphoreType.DMA,
    ],
)
def _embed_sc(table_hbm, idx_hbm, xy_hbm, uv_hbm, out_hbm,
              idx_v, in_v, out_v, xy_v0, xy_v1, uv_v,
              gsem0, gsem1, psem0, psem1, osem0, osem1):
    xy_vs = (xy_v0, xy_v1)
    wid = lax.axis_index("s") * 2 + lax.axis_index("c")
    base_row = wid * ROWS_PER_W
    base_l = wid * (ROWS_PER_W // BATCH)
    gsems = (gsem0, gsem1)
    psems = (psem0, psem1)
    osems = (osem0, osem1)

    pltpu.sync_copy(idx_hbm.at[pl.ds(wid * N_CHUNKS, N_CHUNKS)], idx_v)
    pltpu.sync_copy(uv_hbm, uv_v)

    def start_in(g, s):
        a = wid * N_CHUNKS + g
        pltpu.async_copy(
            xy_hbm.at[pl.ds(a * D_MODEL, D_MODEL)], xy_vs[s], psems[s])
        pltpu.async_copy(table_hbm.at[idx_v.at[g]], in_v.at[s], gsems[s])

    def wait_in(s):
        pltpu.make_async_copy(
            xy_hbm.at[pl.ds(0, D_MODEL)], xy_vs[s], psems[s]).wait()
        pltpu.make_async_copy(
            table_hbm.at[pl.ds(0, CHUNK_ROWS)], in_v.at[s], gsems[s]).wait()

    def wait_out(s):
        pltpu.make_async_copy(
            out_hbm.at[pl.ds(0, L_PER_CHUNK)], out_v.at[s], osems[s]).wait()

    def do_chunk(g, s):
        @pl.when(g + 1 < N_CHUNKS)
        def _():
            start_in(g + 1, 1 - s)

        wait_in(s)

        @pl.when(g >= 2)
        def _():
            wait_out(s)

        def widen(x):
            return (lax.bitcast_convert_type(x << 16, jnp.float32),
                    lax.bitcast_convert_type(x & jnp.int32(-65536),
                                             jnp.float32))

        @plsc.parallel_loop(0, C_PER_ROW // 2)
        def _(p):
            x_a, x_b = widen(xy_vs[s][pl.ds(p * LANES, LANES)])
            y_a, y_b = widen(xy_vs[s][pl.ds(D_MODEL // 2 + p * LANES, LANES)])
            sla = pl.ds(2 * p * LANES, LANES)
            slb = pl.ds((2 * p + 1) * LANES, LANES)

            def load_li(li):
                off = li * (D_MODEL // 2) + p * LANES
                return (uv_v[pl.ds(off, LANES)],
                        uv_v[pl.ds(L_PER_CHUNK * D_MODEL // 2 + off, LANES)],
                        [in_v[s, li * BATCH + b, sla] for b in range(BATCH)],
                        [in_v[s, li * BATCH + b, slb] for b in range(BATCH)])

            grp = {li: load_li(li) for li in range(2)}
            for li in range(L_PER_CHUNK):
                u_i, v_i, ins_a, ins_b = grp.pop(li)
                if li + 2 < L_PER_CHUNK:
                    grp[li + 2] = load_li(li + 2)
                u_a, u_b = widen(u_i)
                v_a, v_b = widen(v_i)
                pe_a = x_a * u_a + y_a * v_a
                pe_b = x_b * u_b + y_b * v_b
                for b in range(BATCH):
                    out_v[s, li, b, sla] = ins_a[b] * SCALE + pe_a
                    out_v[s, li, b, slb] = ins_b[b] * SCALE + pe_b
        l0 = base_l + g * L_PER_CHUNK
        pltpu.async_copy(out_v.at[s], out_hbm.at[pl.ds(l0, L_PER_CHUNK)], osems[s])

    start_in(0, 0)

    def pair_body(i, carry):
        do_chunk(2 * i, 0)
        do_chunk(2 * i + 1, 1)
        return carry

    lax.fori_loop(0, N_CHUNKS // 2, pair_body, 0)
    wait_out(0)
    wait_out(1)


def kernel(encoded_words, embed_weight):
    idx2d = encoded_words.reshape(NW * N_CHUNKS, CHUNK_ROWS)
    xy, uv = _pe_factors()
    return _embed_sc(embed_weight, idx2d, xy, uv)


# speedup vs baseline: 1.0743x; 1.0024x over previous
import functools
import math

import jax
import jax.numpy as jnp
import numpy as np
from jax import lax
from jax.experimental import pallas as pl
from jax.experimental.pallas import tpu as pltpu
from jax.experimental.pallas import tpu_sc as plsc

VOCAB = 100000
D_MODEL = 768
MAX_LEN = 4096
BATCH = 4
SCALE = math.sqrt(D_MODEL)

N_ROWS = MAX_LEN * BATCH
NW = 32
ROWS_PER_W = N_ROWS // NW
CHUNK_ROWS = 32
N_CHUNKS = ROWS_PER_W // CHUNK_ROWS
L_PER_CHUNK = CHUNK_ROWS // BATCH
LANES = 16
C_PER_ROW = D_MODEL // LANES


N_COARSE = MAX_LEN // L_PER_CHUNK


def _pack_bf16_pairs(arr):
    u = np.ascontiguousarray(arr, dtype=np.float32).view(np.uint32)
    bf = ((u + 0x7FFF + ((u >> 16) & 1)) >> 16).astype(np.uint32)
    bb = bf.reshape(arr.shape[0], D_MODEL // 32, 2, LANES)
    return ((bb[:, :, 1, :] << 16) | bb[:, :, 0, :]).reshape(
        arr.shape[0], D_MODEL // 2).view(np.int32)


def _pe_factors():
    w = np.exp(
        np.arange(0, D_MODEL, 2, dtype=np.float32) * (-math.log(10000.0) / D_MODEL)
    )
    a = np.arange(N_COARSE, dtype=np.float32)[:, None]
    d = np.arange(L_PER_CHUNK, dtype=np.float32)[:, None]
    X = np.zeros((N_COARSE, D_MODEL), np.float32)
    Y = np.zeros((N_COARSE, D_MODEL), np.float32)
    X[:, 0::2] = np.sin(L_PER_CHUNK * a * w)
    X[:, 1::2] = np.cos(L_PER_CHUNK * a * w)
    Y[:, 0::2] = np.cos(L_PER_CHUNK * a * w)
    Y[:, 1::2] = -np.sin(L_PER_CHUNK * a * w)
    U = np.zeros((L_PER_CHUNK, D_MODEL), np.float32)
    V = np.zeros((L_PER_CHUNK, D_MODEL), np.float32)
    U[:, 0::2] = np.cos(d * w)
    U[:, 1::2] = np.cos(d * w)
    V[:, 0::2] = np.sin(d * w)
    V[:, 1::2] = np.sin(d * w)
    xy = np.stack([_pack_bf16_pairs(X), _pack_bf16_pairs(Y)], axis=1)
    uv = np.concatenate(
        [_pack_bf16_pairs(U).reshape(-1), _pack_bf16_pairs(V).reshape(-1)])
    return jnp.asarray(xy.reshape(-1)), jnp.asarray(uv)


_MESH = plsc.VectorSubcoreMesh(core_axis_name="c", subcore_axis_name="s")


@functools.partial(
    pl.kernel,
    mesh=_MESH,
    out_type=jax.ShapeDtypeStruct((MAX_LEN, BATCH, D_MODEL), jnp.float32),
    scratch_types=[
        pltpu.VMEM((N_CHUNKS, CHUNK_ROWS), jnp.int32),
        pltpu.VMEM((2, CHUNK_ROWS, D_MODEL), jnp.float32),
        pltpu.VMEM((2, L_PER_CHUNK, BATCH, D_MODEL), jnp.float32),
        pltpu.VMEM((D_MODEL,), jnp.int32),
        pltpu.VMEM((D_MODEL,), jnp.int32),
        pltpu.VMEM((L_PER_CHUNK * D_MODEL,), jnp.int32),
        pltpu.SemaphoreType.DMA,
        pltpu.SemaphoreType.DMA,
        pltpu.SemaphoreType.DMA,
        pltpu.SemaphoreType.DMA,
        pltpu.SemaphoreType.DMA,
        pltpu.SemaphoreType.DMA,
    ],
)
def _embed_sc(table_hbm, idx_hbm, xy_hbm, uv_hbm, out_hbm,
              idx_v, in_v, out_v, xy_v0, xy_v1, uv_v,
              gsem0, gsem1, psem0, psem1, osem0, osem1):
    xy_vs = (xy_v0, xy_v1)
    wid = lax.axis_index("s") * 2 + lax.axis_index("c")
    base_l = wid * (ROWS_PER_W // BATCH)
    gsems = (gsem0, gsem1)
    psems = (psem0, psem1)
    osems = (osem0, osem1)

    pltpu.sync_copy(idx_hbm.at[pl.ds(wid * N_CHUNKS, N_CHUNKS)], idx_v)
    pltpu.sync_copy(uv_hbm, uv_v)

    def start_in(g, s):
        a = wid * N_CHUNKS + g
        pltpu.async_copy(
            xy_hbm.at[pl.ds(a * D_MODEL, D_MODEL)], xy_vs[s], psems[s])
        pltpu.async_copy(table_hbm.at[idx_v.at[g]], in_v.at[s], gsems[s])

    def wait_in(s):
        pltpu.make_async_copy(
            xy_hbm.at[pl.ds(0, D_MODEL)], xy_vs[s], psems[s]).wait()
        pltpu.make_async_copy(
            table_hbm.at[pl.ds(0, CHUNK_ROWS)], in_v.at[s], gsems[s]).wait()

    def wait_out(s):
        pltpu.make_async_copy(
            out_hbm.at[pl.ds(0, L_PER_CHUNK)], out_v.at[s], osems[s]).wait()

    def do_chunk(g, s):
        @pl.when(g + 1 < N_CHUNKS)
        def _():
            start_in(g + 1, 1 - s)

        wait_in(s)

        @pl.when(g >= 2)
        def _():
            wait_out(s)

        def widen(x):
            return (lax.bitcast_convert_type(x << 16, jnp.float32),
                    lax.bitcast_convert_type(x & jnp.int32(-65536),
                                             jnp.float32))

        @plsc.parallel_loop(0, C_PER_ROW // 2)
        def _(p):
            x_a, x_b = widen(xy_vs[s][pl.ds(p * LANES, LANES)])
            y_a, y_b = widen(xy_vs[s][pl.ds(D_MODEL // 2 + p * LANES, LANES)])
            sla = pl.ds(2 * p * LANES, LANES)
            slb = pl.ds((2 * p + 1) * LANES, LANES)

            def load_li(li):
                off = li * (D_MODEL // 2) + p * LANES
                return (uv_v[pl.ds(off, LANES)],
                        uv_v[pl.ds(L_PER_CHUNK * D_MODEL // 2 + off, LANES)],
                        [in_v[s, li * BATCH + b, sla] for b in range(BATCH)],
                        [in_v[s, li * BATCH + b, slb] for b in range(BATCH)])

            grp = {li: load_li(li) for li in range(2)}
            for li in range(L_PER_CHUNK):
                u_i, v_i, ins_a, ins_b = grp.pop(li)
                if li + 2 < L_PER_CHUNK:
                    grp[li + 2] = load_li(li + 2)
                u_a, u_b = widen(u_i)
                v_a, v_b = widen(v_i)
                pe_a = x_a * u_a + y_a * v_a
                pe_b = x_b * u_b + y_b * v_b
                for b in range(BATCH):
                    out_v[s, li, b, sla] = ins_a[b] * SCALE + pe_a
                    out_v[s, li, b, slb] = ins_b[b] * SCALE + pe_b
        l0 = base_l + g * L_PER_CHUNK
        pltpu.async_copy(out_v.at[s], out_hbm.at[pl.ds(l0, L_PER_CHUNK)], osems[s])

    start_in(0, 0)

    def pair_body(i, carry):
        do_chunk(2 * i, 0)
        do_chunk(2 * i + 1, 1)
        return carry

    lax.fori_loop(0, N_CHUNKS // 2, pair_body, 0)
    wait_out(0)
    wait_out(1)


def kernel(encoded_words, embed_weight):
    idx2d = encoded_words.reshape(NW * N_CHUNKS, CHUNK_ROWS)
    xy, uv = _pe_factors()
    return _embed_sc(embed_weight, idx2d, xy, uv)
